# Initial kernel scaffold; baseline (speedup 1.0000x reference)
import jax
import jax.numpy as jnp
from jax.experimental import pallas as pl


def kernel(x, edge_index, W1, b1, W2, b2, Wl, bl):
    raise NotImplementedError("write your pallas kernel here")


# speedup vs baseline: 2.1766x; 2.1766x over previous
import functools

import jax
import jax.numpy as jnp
from jax import lax
from jax.experimental import pallas as pl
from jax.experimental.pallas import tpu as pltpu

N_NODES = 10000
B = 8
C_IN = 128
D_MODEL = 64
F1 = 2 * D_MODEL
NBLK = 2000
NGRID = N_NODES // NBLK


def _conv1_mm_body(x_ref, w1_ref, cnt_ref, ys_ref):
    xb = x_ref[0]
    mm = lax.dot_general(xb, w1_ref[...], (((0,), (0,)), ((), ())),
                         preferred_element_type=jnp.float32)
    dinv = lax.rsqrt(cnt_ref[...] + 1.0)
    ys_ref[0] = dinv * mm


def _stage_b(x, W1, cnt2d):
    return pl.pallas_call(
        _conv1_mm_body,
        grid=(B,),
        in_specs=[
            pl.BlockSpec((1, C_IN, N_NODES), lambda b: (b, 0, 0)),
            pl.BlockSpec((C_IN, F1), lambda b: (0, 0)),
            pl.BlockSpec((N_NODES, 1), lambda b: (0, 0)),
        ],
        out_specs=pl.BlockSpec((1, N_NODES, F1), lambda b: (b, 0, 0)),
        out_shape=jax.ShapeDtypeStruct((B, N_NODES, F1), jnp.float32),
    )(x, W1, cnt2d)


def _stage_d_body(s_ref, ys_ref, cnt_ref, b1_ref, pp_ref, wl_ref, w2_ref,
                  b2_ref, bl_ref, out_ref, t_acc, csum):
    i = pl.program_id(1)

    @pl.when(i == 0)
    def _():
        t_acc[...] = jnp.zeros_like(t_acc)
        csum[...] = jnp.zeros_like(csum)

    dinv = lax.rsqrt(cnt_ref[...] + 1.0)
    z = dinv * (s_ref[0] + ys_ref[0]) + b1_ref[...]
    g = jnp.where(z >= 0.0, z, 0.01 * z)
    p = dinv * pp_ref[...] + (dinv * dinv) * wl_ref[...]
    t_acc[...] += lax.dot_general(g, p, (((0,), (0,)), ((), ())),
                                  preferred_element_type=jnp.float32)
    csum[...] += jnp.sum(wl_ref[...], axis=0, keepdims=True)

    @pl.when(i == NGRID - 1)
    def _():
        w2t = lax.dot_general(w2_ref[...], t_acc[...],
                              (((0,), (0,)), ((), ())),
                              preferred_element_type=jnp.float32)
        out_ref[0] = w2t + b2_ref[...] * csum[...] + bl_ref[...]


def _stage_d(S, Ys, cnt2d, b1, Pp, Wl, W2, b2, bl):
    return pl.pallas_call(
        _stage_d_body,
        grid=(B, NGRID),
        in_specs=[
            pl.BlockSpec((1, NBLK, F1), lambda b, i: (b, i, 0)),
            pl.BlockSpec((1, NBLK, F1), lambda b, i: (b, i, 0)),
            pl.BlockSpec((NBLK, 1), lambda b, i: (i, 0)),
            pl.BlockSpec((1, F1), lambda b, i: (0, 0)),
            pl.BlockSpec((NBLK, D_MODEL), lambda b, i: (i, 0)),
            pl.BlockSpec((NBLK, D_MODEL), lambda b, i: (i, 0)),
            pl.BlockSpec((F1, C_IN), lambda b, i: (0, 0)),
            pl.BlockSpec((C_IN, 1), lambda b, i: (0, 0)),
            pl.BlockSpec((1, D_MODEL), lambda b, i: (0, 0)),
        ],
        out_specs=pl.BlockSpec((1, C_IN, D_MODEL), lambda b, i: (b, 0, 0)),
        out_shape=jax.ShapeDtypeStruct((B, C_IN, D_MODEL), jnp.float32),
        scratch_shapes=[
            pltpu.VMEM((F1, D_MODEL), jnp.float32),
            pltpu.VMEM((1, D_MODEL), jnp.float32),
        ],
    )(S, Ys, cnt2d, b1, Pp, Wl, W2, b2, bl)


def kernel(x, edge_index, W1, b1, W2, b2, Wl, bl):
    src = edge_index[0]
    dst = edge_index[1]
    cnt = jnp.zeros((N_NODES,), jnp.float32).at[dst].add(1.0)
    cnt2d = cnt[:, None]
    dinv = lax.rsqrt(cnt + 1.0)

    Ys = _stage_b(x, W1, cnt2d)

    S = jnp.zeros_like(Ys).at[:, dst, :].add(Ys[:, src, :])

    Wls = dinv[:, None] * Wl
    Pp = jnp.zeros_like(Wls).at[src, :].add(Wls[dst, :])

    return _stage_d(S, Ys, cnt2d, b1[None, :], Pp, Wl, W2, b2[:, None],
                    bl[None, :])


# speedup vs baseline: 47.3592x; 21.7586x over previous
import functools

import jax
import jax.numpy as jnp
from jax import lax
from jax.experimental import pallas as pl
from jax.experimental.pallas import tpu as pltpu
from jax.experimental.pallas import tpu_sc as plsc

N_NODES = 10000
B = 8
C_IN = 128
D_MODEL = 64
F1 = 2 * D_MODEL
NBLK = 2000
NGRID = N_NODES // NBLK

E = 320000
NC = 2
NS = 16
KW = 125
EROWS = E // KW
ZROWS = EROWS // NS
PROWS = EROWS // (NC * NS)
OWN = 640
NACC = NS * OWN
LAST_OWN = N_NODES - (NS - 1) * OWN
CHUNKS_PER_SC = B // NC

def _sc_mesh():
    return plsc.VectorSubcoreMesh(core_axis_name="c", subcore_axis_name="s",
                                  num_cores=NC, num_subcores=NS)


def _fill_const(buf, rows, val):
    def _f(i, _):
        for j in range(F1 // 16):
            buf[i, pl.ds(j * 16, 16)] = jnp.full((16,), val, jnp.float32)
        return _

    lax.fori_loop(0, rows, _f, None)


def _copy_out_slice(s, acc, dst_hbm):
    @pl.when(s < NS - 1)
    def _():
        pltpu.sync_copy(acc.at[pl.ds(s * OWN, OWN)],
                        dst_hbm.at[pl.ds(s * OWN, OWN)])

    @pl.when(s == NS - 1)
    def _():
        pltpu.sync_copy(acc.at[pl.ds((NS - 1) * OWN, LAST_OWN)],
                        dst_hbm.at[pl.ds((NS - 1) * OWN, LAST_OWN)])


def _sc_deg_body(dst2d, degp, acc, dstv, onesb, zerob, sem):
    c = lax.axis_index("c")
    s = lax.axis_index("s")

    _fill_const(onesb, KW, 1.0)
    _fill_const(zerob, 64, 0.0)
    pltpu.sync_copy(dst2d.at[pl.ds((c * NS + s) * PROWS, PROWS)], dstv)
    for j in range(OWN // 64):
        pltpu.sync_copy(zerob, acc.at[pl.ds(s * OWN + j * 64, 64)])
    plsc.subcore_barrier()

    def _step(i, _):
        pltpu.sync_copy(onesb, acc.at[dstv.at[i]], add=True)
        return _

    lax.fori_loop(0, PROWS, _step, None)
    plsc.subcore_barrier()
    _copy_out_slice(s, acc, degp.at[c])


def _sc_deg(dst2d):
    return pl.kernel(
        _sc_deg_body,
        out_type=jax.ShapeDtypeStruct((NC, N_NODES, F1), jnp.float32),
        mesh=_sc_mesh(),
        scratch_types=[
            pltpu.VMEM_SHARED((NACC, F1), jnp.float32),
            pltpu.VMEM((PROWS, KW), jnp.int32),
            pltpu.VMEM((KW, F1), jnp.float32),
            pltpu.VMEM((64, F1), jnp.float32),
            pltpu.SemaphoreType.DMA,
        ],
    )(dst2d)


def _sc_scatter_body(src2d, dst2d, ys, wls, s_out, pp_out,
                     acc, gsrc, gdst, gbuf, zerob, sem):
    c = lax.axis_index("c")
    s = lax.axis_index("s")

    _fill_const(zerob, 64, 0.0)

    def _zero_acc():
        for j in range(OWN // 64):
            pltpu.sync_copy(zerob, acc.at[pl.ds(s * OWN + j * 64, 64)])

    p0 = (c * NS + s) * PROWS
    pltpu.sync_copy(src2d.at[pl.ds(p0, PROWS)], gsrc)
    pltpu.sync_copy(dst2d.at[pl.ds(p0, PROWS)], gdst)
    _zero_acc()
    plsc.subcore_barrier()

    def _pstep(i, _):
        pltpu.async_copy(wls.at[gdst.at[i]], gbuf, sem).wait()
        pltpu.sync_copy(gbuf, acc.at[gsrc.at[i]], add=True)
        return _

    lax.fori_loop(0, PROWS, _pstep, None)
    plsc.subcore_barrier()
    _copy_out_slice(s, acc, pp_out.at[c])

    for q in range(CHUNKS_PER_SC):
        chunk = c * CHUNKS_PER_SC + q
        _zero_acc()
        plsc.subcore_barrier()
        for h in range(ZROWS // PROWS):
            z0 = s * ZROWS + h * PROWS
            pltpu.sync_copy(src2d.at[pl.ds(z0, PROWS)], gsrc)
            pltpu.sync_copy(dst2d.at[pl.ds(z0, PROWS)], gdst)

            def _zstep(i, _):
                pltpu.async_copy(ys.at[chunk].at[gsrc.at[i]], gbuf, sem).wait()
                pltpu.sync_copy(gbuf, acc.at[gdst.at[i]], add=True)
                return _

            lax.fori_loop(0, PROWS, _zstep, None)
        plsc.subcore_barrier()
        _copy_out_slice(s, acc, s_out.at[chunk])


def _sc_scatter(src2d, dst2d, Ys, Wls):
    return pl.kernel(
        _sc_scatter_body,
        out_type=(
            jax.ShapeDtypeStruct((B, N_NODES, F1), jnp.float32),
            jax.ShapeDtypeStruct((NC, N_NODES, F1), jnp.float32),
        ),
        mesh=_sc_mesh(),
        scratch_types=[
            pltpu.VMEM_SHARED((NACC, F1), jnp.float32),
            pltpu.VMEM((PROWS, KW), jnp.int32),
            pltpu.VMEM((PROWS, KW), jnp.int32),
            pltpu.VMEM((KW, F1), jnp.float32),
            pltpu.VMEM((64, F1), jnp.float32),
            pltpu.SemaphoreType.DMA,
        ],
    )(src2d, dst2d, Ys, Wls)


def _conv1_mm_body(x_ref, w1_ref, wl_ref, degp_ref, ys_ref, wls_ref):
    xb = x_ref[0]
    mm = lax.dot_general(xb, w1_ref[...], (((0,), (0,)), ((), ())),
                         preferred_element_type=jnp.float32)
    cnt = degp_ref[0, :, 0:1] + degp_ref[1, :, 0:1]
    dinv = lax.rsqrt(cnt + 1.0)
    ys_ref[0] = dinv * mm
    wls_ref[...] = jnp.concatenate(
        [dinv * wl_ref[...], jnp.zeros((NBLK * NGRID, F1 - D_MODEL), jnp.float32)],
        axis=1)


def _stage_b(x, W1, Wl, degp):
    return pl.pallas_call(
        _conv1_mm_body,
        grid=(B,),
        in_specs=[
            pl.BlockSpec((1, C_IN, N_NODES), lambda b: (b, 0, 0)),
            pl.BlockSpec((C_IN, F1), lambda b: (0, 0)),
            pl.BlockSpec((N_NODES, D_MODEL), lambda b: (0, 0)),
            pl.BlockSpec((NC, N_NODES, F1), lambda b: (0, 0, 0)),
        ],
        out_specs=[
            pl.BlockSpec((1, N_NODES, F1), lambda b: (b, 0, 0)),
            pl.BlockSpec((N_NODES, F1), lambda b: (0, 0)),
        ],
        out_shape=[
            jax.ShapeDtypeStruct((B, N_NODES, F1), jnp.float32),
            jax.ShapeDtypeStruct((N_NODES, F1), jnp.float32),
        ],
    )(x, W1, Wl, degp)


def _stage_d_body(s_ref, ys_ref, degp_ref, b1_ref, pp_ref, wl_ref, w2_ref,
                  b2_ref, bl_ref, out_ref, t_acc, csum):
    i = pl.program_id(1)

    @pl.when(i == 0)
    def _():
        t_acc[...] = jnp.zeros_like(t_acc)
        csum[...] = jnp.zeros_like(csum)

    cnt = degp_ref[0, :, 0:1] + degp_ref[1, :, 0:1]
    dinv = lax.rsqrt(cnt + 1.0)
    z = dinv * (s_ref[0] + ys_ref[0]) + b1_ref[...]
    g = jnp.where(z >= 0.0, z, 0.01 * z)
    pp = pp_ref[0, :, 0:D_MODEL] + pp_ref[1, :, 0:D_MODEL]
    p = dinv * pp + (dinv * dinv) * wl_ref[...]
    t_acc[...] += lax.dot_general(g, p, (((0,), (0,)), ((), ())),
                                  preferred_element_type=jnp.float32)
    csum[...] += jnp.sum(wl_ref[...], axis=0, keepdims=True)

    @pl.when(i == NGRID - 1)
    def _():
        w2t = lax.dot_general(w2_ref[...], t_acc[...],
                              (((0,), (0,)), ((), ())),
                              preferred_element_type=jnp.float32)
        out_ref[0] = w2t + b2_ref[...] * csum[...] + bl_ref[...]


def _stage_d(S, Ys, degp, b1, Pp, Wl, W2, b2, bl):
    return pl.pallas_call(
        _stage_d_body,
        grid=(B, NGRID),
        in_specs=[
            pl.BlockSpec((1, NBLK, F1), lambda b, i: (b, i, 0)),
            pl.BlockSpec((1, NBLK, F1), lambda b, i: (b, i, 0)),
            pl.BlockSpec((NC, NBLK, F1), lambda b, i: (0, i, 0)),
            pl.BlockSpec((1, F1), lambda b, i: (0, 0)),
            pl.BlockSpec((NC, NBLK, F1), lambda b, i: (0, i, 0)),
            pl.BlockSpec((NBLK, D_MODEL), lambda b, i: (i, 0)),
            pl.BlockSpec((F1, C_IN), lambda b, i: (0, 0)),
            pl.BlockSpec((C_IN, 1), lambda b, i: (0, 0)),
            pl.BlockSpec((1, D_MODEL), lambda b, i: (0, 0)),
        ],
        out_specs=pl.BlockSpec((1, C_IN, D_MODEL), lambda b, i: (b, 0, 0)),
        out_shape=jax.ShapeDtypeStruct((B, C_IN, D_MODEL), jnp.float32),
        scratch_shapes=[
            pltpu.VMEM((F1, D_MODEL), jnp.float32),
            pltpu.VMEM((1, D_MODEL), jnp.float32),
        ],
    )(S, Ys, degp, b1, Pp, Wl, W2, b2, bl)


def kernel(x, edge_index, W1, b1, W2, b2, Wl, bl):
    src2d = edge_index[0].reshape(EROWS, KW)
    dst2d = edge_index[1].reshape(EROWS, KW)

    degp = _sc_deg(dst2d)
    Ys, Wls = _stage_b(x, W1, Wl, degp)
    S, Pp = _sc_scatter(src2d, dst2d, Ys, Wls)
    return _stage_d(S, Ys, degp, b1[None, :], Pp, Wl, W2, b2[:, None],
                    bl[None, :])


# speedup vs baseline: 60.5811x; 1.2792x over previous
import functools

import jax
import jax.numpy as jnp
from jax import lax
from jax.experimental import pallas as pl
from jax.experimental.pallas import tpu as pltpu
from jax.experimental.pallas import tpu_sc as plsc

N_NODES = 10000
B = 8
C_IN = 128
D_MODEL = 64
F1 = 2 * D_MODEL
NBLK = 2000
NGRID = N_NODES // NBLK

E = 320000
NC = 2
NS = 16
KW = 125
EROWS = E // KW
ZROWS = EROWS // NS
PROWS = EROWS // (NC * NS)
SB = 40
OWN = 640
NACC = NS * OWN
LAST_OWN = N_NODES - (NS - 1) * OWN
CHUNKS_PER_SC = B // NC

def _sc_mesh():
    return plsc.VectorSubcoreMesh(core_axis_name="c", subcore_axis_name="s",
                                  num_cores=NC, num_subcores=NS)


def _fill_const(buf, rows, val):
    def _f(i, _):
        for j in range(F1 // 16):
            buf[i, pl.ds(j * 16, 16)] = jnp.full((16,), val, jnp.float32)
        return _

    lax.fori_loop(0, rows, _f, None)


def _copy_out_slice(s, acc, dst_hbm):
    @pl.when(s < NS - 1)
    def _():
        pltpu.sync_copy(acc.at[pl.ds(s * OWN, OWN)],
                        dst_hbm.at[pl.ds(s * OWN, OWN)])

    @pl.when(s == NS - 1)
    def _():
        pltpu.sync_copy(acc.at[pl.ds((NS - 1) * OWN, LAST_OWN)],
                        dst_hbm.at[pl.ds((NS - 1) * OWN, LAST_OWN)])


def _sc_deg_body(dst2d, degp, acc, dstv, onesb, zerob, sem):
    c = lax.axis_index("c")
    s = lax.axis_index("s")

    _fill_const(onesb, KW, 1.0)
    _fill_const(zerob, 64, 0.0)
    pltpu.sync_copy(dst2d.at[pl.ds((c * NS + s) * PROWS, PROWS)], dstv)
    for j in range(OWN // 64):
        pltpu.sync_copy(zerob, acc.at[pl.ds(s * OWN + j * 64, 64)])
    plsc.subcore_barrier()

    def _step(i, _):
        pltpu.sync_copy(onesb, acc.at[dstv.at[i]], add=True)
        return _

    lax.fori_loop(0, PROWS, _step, None)
    plsc.subcore_barrier()
    _copy_out_slice(s, acc, degp.at[c])


def _sc_deg(dst2d):
    return pl.kernel(
        _sc_deg_body,
        out_type=jax.ShapeDtypeStruct((NC, N_NODES, F1), jnp.float32),
        mesh=_sc_mesh(),
        scratch_types=[
            pltpu.VMEM_SHARED((NACC, F1), jnp.float32),
            pltpu.VMEM((PROWS, KW), jnp.int32),
            pltpu.VMEM((KW, F1), jnp.float32),
            pltpu.VMEM((64, F1), jnp.float32),
            pltpu.SemaphoreType.DMA,
        ],
    )(dst2d)


def _sc_scatter_body(src2d, dst2d, ys, wls, s_out, pp_out,
                     acc, gsrc, gdst, gbuf0, gbuf1, sem0, sem1):
    c = lax.axis_index("c")
    s = lax.axis_index("s")

    def _zero_acc():
        _fill_const(gbuf0, 64, 0.0)
        for j in range(OWN // 64):
            pltpu.sync_copy(gbuf0.at[pl.ds(0, 64)],
                            acc.at[pl.ds(s * OWN + j * 64, 64)])

    def _pipe_block(table, g2d, s2d, r0):
        pltpu.sync_copy(g2d.at[pl.ds(r0, SB)], gsrc)
        pltpu.sync_copy(s2d.at[pl.ds(r0, SB)], gdst)
        pltpu.async_copy(table.at[gsrc.at[0]], gbuf0, sem0)

        def _pair(j, _):
            i0 = j * 2
            pltpu.make_async_copy(table.at[gsrc.at[0]], gbuf0, sem0).wait()
            pltpu.async_copy(table.at[gsrc.at[i0 + 1]], gbuf1, sem1)
            pltpu.sync_copy(gbuf0, acc.at[gdst.at[i0]], add=True)
            pltpu.make_async_copy(table.at[gsrc.at[0]], gbuf1, sem1).wait()
            nxt = jnp.minimum(i0 + 2, SB - 1)
            pltpu.async_copy(table.at[gsrc.at[nxt]], gbuf0, sem0)
            pltpu.sync_copy(gbuf1, acc.at[gdst.at[i0 + 1]], add=True)
            return _

        lax.fori_loop(0, SB // 2, _pair, None)
        pltpu.make_async_copy(table.at[gsrc.at[0]], gbuf0, sem0).wait()

    _zero_acc()
    plsc.subcore_barrier()
    p0 = (c * NS + s) * PROWS
    for h in range(PROWS // SB):
        _pipe_block(wls, dst2d, src2d, p0 + h * SB)
    plsc.subcore_barrier()
    _copy_out_slice(s, acc, pp_out.at[c])

    for q in range(CHUNKS_PER_SC):
        chunk = c * CHUNKS_PER_SC + q
        _zero_acc()
        plsc.subcore_barrier()
        for h in range(ZROWS // SB):
            _pipe_block(ys.at[chunk], src2d, dst2d, s * ZROWS + h * SB)
        plsc.subcore_barrier()
        _copy_out_slice(s, acc, s_out.at[chunk])


def _sc_scatter(src2d, dst2d, Ys, Wls):
    return pl.kernel(
        _sc_scatter_body,
        out_type=(
            jax.ShapeDtypeStruct((B, N_NODES, F1), jnp.float32),
            jax.ShapeDtypeStruct((NC, N_NODES, F1), jnp.float32),
        ),
        mesh=_sc_mesh(),
        scratch_types=[
            pltpu.VMEM_SHARED((NACC, F1), jnp.float32),
            pltpu.VMEM((SB, KW), jnp.int32),
            pltpu.VMEM((SB, KW), jnp.int32),
            pltpu.VMEM((KW, F1), jnp.float32),
            pltpu.VMEM((KW, F1), jnp.float32),
            pltpu.SemaphoreType.DMA,
            pltpu.SemaphoreType.DMA,
        ],
    )(src2d, dst2d, Ys, Wls)


def _conv1_mm_body(x_ref, w1_ref, wl_ref, degp_ref, ys_ref, wls_ref):
    xb = x_ref[0]
    mm = lax.dot_general(xb, w1_ref[...], (((0,), (0,)), ((), ())),
                         preferred_element_type=jnp.float32)
    cnt = degp_ref[0, :, 0:1] + degp_ref[1, :, 0:1]
    dinv = lax.rsqrt(cnt + 1.0)
    ys_ref[0] = dinv * mm
    wls_ref[...] = jnp.concatenate(
        [dinv * wl_ref[...], jnp.zeros((NBLK * NGRID, F1 - D_MODEL), jnp.float32)],
        axis=1)


def _stage_b(x, W1, Wl, degp):
    return pl.pallas_call(
        _conv1_mm_body,
        grid=(B,),
        in_specs=[
            pl.BlockSpec((1, C_IN, N_NODES), lambda b: (b, 0, 0)),
            pl.BlockSpec((C_IN, F1), lambda b: (0, 0)),
            pl.BlockSpec((N_NODES, D_MODEL), lambda b: (0, 0)),
            pl.BlockSpec((NC, N_NODES, F1), lambda b: (0, 0, 0)),
        ],
        out_specs=[
            pl.BlockSpec((1, N_NODES, F1), lambda b: (b, 0, 0)),
            pl.BlockSpec((N_NODES, F1), lambda b: (0, 0)),
        ],
        out_shape=[
            jax.ShapeDtypeStruct((B, N_NODES, F1), jnp.float32),
            jax.ShapeDtypeStruct((N_NODES, F1), jnp.float32),
        ],
    )(x, W1, Wl, degp)


def _stage_d_body(s_ref, ys_ref, degp_ref, b1_ref, pp_ref, wl_ref, w2_ref,
                  b2_ref, bl_ref, out_ref, t_acc, csum):
    i = pl.program_id(1)

    @pl.when(i == 0)
    def _():
        t_acc[...] = jnp.zeros_like(t_acc)
        csum[...] = jnp.zeros_like(csum)

    cnt = degp_ref[0, :, 0:1] + degp_ref[1, :, 0:1]
    dinv = lax.rsqrt(cnt + 1.0)
    z = dinv * (s_ref[0] + ys_ref[0]) + b1_ref[...]
    g = jnp.where(z >= 0.0, z, 0.01 * z)
    pp = pp_ref[0, :, 0:D_MODEL] + pp_ref[1, :, 0:D_MODEL]
    p = dinv * pp + (dinv * dinv) * wl_ref[...]
    t_acc[...] += lax.dot_general(g, p, (((0,), (0,)), ((), ())),
                                  preferred_element_type=jnp.float32)
    csum[...] += jnp.sum(wl_ref[...], axis=0, keepdims=True)

    @pl.when(i == NGRID - 1)
    def _():
        w2t = lax.dot_general(w2_ref[...], t_acc[...],
                              (((0,), (0,)), ((), ())),
                              preferred_element_type=jnp.float32)
        out_ref[0] = w2t + b2_ref[...] * csum[...] + bl_ref[...]


def _stage_d(S, Ys, degp, b1, Pp, Wl, W2, b2, bl):
    return pl.pallas_call(
        _stage_d_body,
        grid=(B, NGRID),
        in_specs=[
            pl.BlockSpec((1, NBLK, F1), lambda b, i: (b, i, 0)),
            pl.BlockSpec((1, NBLK, F1), lambda b, i: (b, i, 0)),
            pl.BlockSpec((NC, NBLK, F1), lambda b, i: (0, i, 0)),
            pl.BlockSpec((1, F1), lambda b, i: (0, 0)),
            pl.BlockSpec((NC, NBLK, F1), lambda b, i: (0, i, 0)),
            pl.BlockSpec((NBLK, D_MODEL), lambda b, i: (i, 0)),
            pl.BlockSpec((F1, C_IN), lambda b, i: (0, 0)),
            pl.BlockSpec((C_IN, 1), lambda b, i: (0, 0)),
            pl.BlockSpec((1, D_MODEL), lambda b, i: (0, 0)),
        ],
        out_specs=pl.BlockSpec((1, C_IN, D_MODEL), lambda b, i: (b, 0, 0)),
        out_shape=jax.ShapeDtypeStruct((B, C_IN, D_MODEL), jnp.float32),
        scratch_shapes=[
            pltpu.VMEM((F1, D_MODEL), jnp.float32),
            pltpu.VMEM((1, D_MODEL), jnp.float32),
        ],
    )(S, Ys, degp, b1, Pp, Wl, W2, b2, bl)


def kernel(x, edge_index, W1, b1, W2, b2, Wl, bl):
    src2d = edge_index[0].reshape(EROWS, KW)
    dst2d = edge_index[1].reshape(EROWS, KW)

    degp = _sc_deg(dst2d)
    Ys, Wls = _stage_b(x, W1, Wl, degp)
    S, Pp = _sc_scatter(src2d, dst2d, Ys, Wls)
    return _stage_d(S, Ys, degp, b1[None, :], Pp, Wl, W2, b2[:, None],
                    bl[None, :])


# speedup vs baseline: 75.5740x; 1.2475x over previous
import functools

import jax
import jax.numpy as jnp
from jax import lax
from jax.experimental import pallas as pl
from jax.experimental.pallas import tpu as pltpu
from jax.experimental.pallas import tpu_sc as plsc

N_NODES = 10000
B = 8
C_IN = 128
D_MODEL = 64
F1 = 2 * D_MODEL
NBLK = 2000
NGRID = N_NODES // NBLK

E = 320000
NC = 2
NS = 16
KW = 100
EROWS = E // KW
ZROWS = EROWS // NS
SB = 40
KW_DEG = 125
EROWS_DEG = E // KW_DEG
PROWS_DEG = EROWS_DEG // (NC * NS)
OWN = 624
NACC = N_NODES
LAST_OWN = N_NODES - (NS - 1) * OWN
CHUNKS_PER_SC = B // NC

def _sc_mesh():
    return plsc.VectorSubcoreMesh(core_axis_name="c", subcore_axis_name="s",
                                  num_cores=NC, num_subcores=NS)


def _fill_const(buf, rows, val):
    def _f(i, _):
        for j in range(F1 // 16):
            buf[i, pl.ds(j * 16, 16)] = jnp.full((16,), val, jnp.float32)
        return _

    lax.fori_loop(0, rows, _f, None)


def _copy_out_slice(s, acc, dst_hbm):
    @pl.when(s < NS - 1)
    def _():
        pltpu.sync_copy(acc.at[pl.ds(s * OWN, OWN)],
                        dst_hbm.at[pl.ds(s * OWN, OWN)])

    @pl.when(s == NS - 1)
    def _():
        pltpu.sync_copy(acc.at[pl.ds((NS - 1) * OWN, LAST_OWN)],
                        dst_hbm.at[pl.ds((NS - 1) * OWN, LAST_OWN)])


def _zero_own(s, acc, zsrc48, zsrc16):
    for j in range(OWN // 48):
        pltpu.sync_copy(zsrc48, acc.at[pl.ds(s * OWN + j * 48, 48)])

    @pl.when(s == NS - 1)
    def _():
        pltpu.sync_copy(zsrc16, acc.at[pl.ds((NS - 1) * OWN + OWN, 16)])


def _sc_deg_body(dst2d, degp, acc, dstv, onesb, zerob, sem):
    c = lax.axis_index("c")
    s = lax.axis_index("s")

    _fill_const(onesb, KW_DEG, 1.0)
    _fill_const(zerob, 48, 0.0)
    pltpu.sync_copy(dst2d.at[pl.ds((c * NS + s) * PROWS_DEG, PROWS_DEG)], dstv)
    _zero_own(s, acc, zerob, zerob.at[pl.ds(0, 16)])
    plsc.subcore_barrier()

    def _step(i, _):
        pltpu.sync_copy(onesb, acc.at[dstv.at[i]], add=True)
        return _

    lax.fori_loop(0, PROWS_DEG, _step, None)
    plsc.subcore_barrier()
    _copy_out_slice(s, acc, degp.at[c])


def _sc_deg(dst2d):
    return pl.kernel(
        _sc_deg_body,
        out_type=jax.ShapeDtypeStruct((NC, N_NODES, F1), jnp.float32),
        mesh=_sc_mesh(),
        scratch_types=[
            pltpu.VMEM_SHARED((NACC, F1), jnp.float32),
            pltpu.VMEM((PROWS_DEG, KW_DEG), jnp.int32),
            pltpu.VMEM((KW_DEG, F1), jnp.float32),
            pltpu.VMEM((48, F1), jnp.float32),
            pltpu.SemaphoreType.DMA,
        ],
    )(dst2d)


def _sc_scatter_body(src2d, dst2d, ys, wls, s_out, pp_out,
                     acc, gsrc, gdst, gbuf0, gbuf1, gbuf2, sem0, sem1, sem2):
    c = lax.axis_index("c")
    s = lax.axis_index("s")

    def _zero_acc():
        _fill_const(gbuf0, 48, 0.0)
        _zero_own(s, acc, gbuf0.at[pl.ds(0, 48)], gbuf0.at[pl.ds(0, 16)])

    def _pipe_block(table, g2d, s2d, r0):
        bufs = ((gbuf0, sem0), (gbuf1, sem1), (gbuf2, sem2))
        pltpu.sync_copy(g2d.at[pl.ds(r0, SB)], gsrc)
        pltpu.sync_copy(s2d.at[pl.ds(r0, SB)], gdst)
        for k in range(3):
            pltpu.async_copy(table.at[gsrc.at[k]], bufs[k][0], bufs[k][1])

        def _tri(j, _):
            i0 = j * 3
            for k in range(3):
                buf, sem = bufs[k]
                pltpu.make_async_copy(table.at[gsrc.at[0]], buf, sem).wait()
                pltpu.sync_copy(buf, acc.at[gdst.at[i0 + k]], add=True)
                nxt = jnp.minimum(i0 + k + 3, SB - 1)
                pltpu.async_copy(table.at[gsrc.at[nxt]], buf, sem)
            return _

        lax.fori_loop(0, SB // 3, _tri, None)
        for t in range(3 * (SB // 3), SB):
            buf, sem = bufs[t % 3]
            pltpu.make_async_copy(table.at[gsrc.at[0]], buf, sem).wait()
            pltpu.sync_copy(buf, acc.at[gdst.at[t]], add=True)
        for u in range(3 - (SB - 3 * (SB // 3))):
            buf, sem = bufs[(SB + u) % 3]
            pltpu.make_async_copy(table.at[gsrc.at[0]], buf, sem).wait()

    _zero_acc()
    plsc.subcore_barrier()
    w = c * NS + s

    @pl.when(w < NS)
    def _():
        for h in range(3):
            _pipe_block(wls, dst2d, src2d, w * (3 * SB) + h * SB)

    @pl.when(w >= NS)
    def _():
        for h in range(2):
            _pipe_block(wls, dst2d, src2d,
                        NS * (3 * SB) + (w - NS) * (2 * SB) + h * SB)

    plsc.subcore_barrier()
    _copy_out_slice(s, acc, pp_out.at[c])

    for q in range(CHUNKS_PER_SC):
        chunk = c * CHUNKS_PER_SC + q
        _zero_acc()
        plsc.subcore_barrier()
        for h in range(ZROWS // SB):
            _pipe_block(ys.at[chunk], src2d, dst2d, s * ZROWS + h * SB)
        plsc.subcore_barrier()
        _copy_out_slice(s, acc, s_out.at[chunk])


def _sc_scatter(src2d, dst2d, Ys, Wls):
    return pl.kernel(
        _sc_scatter_body,
        out_type=(
            jax.ShapeDtypeStruct((B, N_NODES, F1), jnp.float32),
            jax.ShapeDtypeStruct((NC, N_NODES, F1), jnp.float32),
        ),
        mesh=_sc_mesh(),
        scratch_types=[
            pltpu.VMEM_SHARED((NACC, F1), jnp.float32),
            pltpu.VMEM((SB, KW), jnp.int32),
            pltpu.VMEM((SB, KW), jnp.int32),
            pltpu.VMEM((KW, F1), jnp.float32),
            pltpu.VMEM((KW, F1), jnp.float32),
            pltpu.VMEM((KW, F1), jnp.float32),
            pltpu.SemaphoreType.DMA,
            pltpu.SemaphoreType.DMA,
            pltpu.SemaphoreType.DMA,
        ],
    )(src2d, dst2d, Ys, Wls)


def _conv1_mm_body(x_ref, w1_ref, wl_ref, degp_ref, ys_ref, wls_ref):
    xb = x_ref[0]
    mm = lax.dot_general(xb, w1_ref[...], (((0,), (0,)), ((), ())),
                         preferred_element_type=jnp.float32)
    cnt = degp_ref[0, :, 0:1] + degp_ref[1, :, 0:1]
    dinv = lax.rsqrt(cnt + 1.0)
    ys_ref[0] = dinv * mm
    wls_ref[...] = jnp.concatenate(
        [dinv * wl_ref[...], jnp.zeros((NBLK * NGRID, F1 - D_MODEL), jnp.float32)],
        axis=1)


def _stage_b(x, W1, Wl, degp):
    return pl.pallas_call(
        _conv1_mm_body,
        grid=(B,),
        in_specs=[
            pl.BlockSpec((1, C_IN, N_NODES), lambda b: (b, 0, 0)),
            pl.BlockSpec((C_IN, F1), lambda b: (0, 0)),
            pl.BlockSpec((N_NODES, D_MODEL), lambda b: (0, 0)),
            pl.BlockSpec((NC, N_NODES, F1), lambda b: (0, 0, 0)),
        ],
        out_specs=[
            pl.BlockSpec((1, N_NODES, F1), lambda b: (b, 0, 0)),
            pl.BlockSpec((N_NODES, F1), lambda b: (0, 0)),
        ],
        out_shape=[
            jax.ShapeDtypeStruct((B, N_NODES, F1), jnp.float32),
            jax.ShapeDtypeStruct((N_NODES, F1), jnp.float32),
        ],
    )(x, W1, Wl, degp)


def _stage_d_body(s_ref, ys_ref, degp_ref, b1_ref, pp_ref, wl_ref, w2_ref,
                  b2_ref, bl_ref, out_ref, t_acc, csum):
    i = pl.program_id(1)

    @pl.when(i == 0)
    def _():
        t_acc[...] = jnp.zeros_like(t_acc)
        csum[...] = jnp.zeros_like(csum)

    cnt = degp_ref[0, :, 0:1] + degp_ref[1, :, 0:1]
    dinv = lax.rsqrt(cnt + 1.0)
    z = dinv * (s_ref[0] + ys_ref[0]) + b1_ref[...]
    g = jnp.where(z >= 0.0, z, 0.01 * z)
    pp = pp_ref[0, :, 0:D_MODEL] + pp_ref[1, :, 0:D_MODEL]
    p = dinv * pp + (dinv * dinv) * wl_ref[...]
    t_acc[...] += lax.dot_general(g, p, (((0,), (0,)), ((), ())),
                                  preferred_element_type=jnp.float32)
    csum[...] += jnp.sum(wl_ref[...], axis=0, keepdims=True)

    @pl.when(i == NGRID - 1)
    def _():
        w2t = lax.dot_general(w2_ref[...], t_acc[...],
                              (((0,), (0,)), ((), ())),
                              preferred_element_type=jnp.float32)
        out_ref[0] = w2t + b2_ref[...] * csum[...] + bl_ref[...]


def _stage_d(S, Ys, degp, b1, Pp, Wl, W2, b2, bl):
    return pl.pallas_call(
        _stage_d_body,
        grid=(B, NGRID),
        in_specs=[
            pl.BlockSpec((1, NBLK, F1), lambda b, i: (b, i, 0)),
            pl.BlockSpec((1, NBLK, F1), lambda b, i: (b, i, 0)),
            pl.BlockSpec((NC, NBLK, F1), lambda b, i: (0, i, 0)),
            pl.BlockSpec((1, F1), lambda b, i: (0, 0)),
            pl.BlockSpec((NC, NBLK, F1), lambda b, i: (0, i, 0)),
            pl.BlockSpec((NBLK, D_MODEL), lambda b, i: (i, 0)),
            pl.BlockSpec((F1, C_IN), lambda b, i: (0, 0)),
            pl.BlockSpec((C_IN, 1), lambda b, i: (0, 0)),
            pl.BlockSpec((1, D_MODEL), lambda b, i: (0, 0)),
        ],
        out_specs=pl.BlockSpec((1, C_IN, D_MODEL), lambda b, i: (b, 0, 0)),
        out_shape=jax.ShapeDtypeStruct((B, C_IN, D_MODEL), jnp.float32),
        scratch_shapes=[
            pltpu.VMEM((F1, D_MODEL), jnp.float32),
            pltpu.VMEM((1, D_MODEL), jnp.float32),
        ],
    )(S, Ys, degp, b1, Pp, Wl, W2, b2, bl)


def kernel(x, edge_index, W1, b1, W2, b2, Wl, bl):
    src2d = edge_index[0].reshape(EROWS, KW)
    dst2d = edge_index[1].reshape(EROWS, KW)
    dst2d_deg = edge_index[1].reshape(EROWS_DEG, KW_DEG)

    degp = _sc_deg(dst2d_deg)
    Ys, Wls = _stage_b(x, W1, Wl, degp)
    S, Pp = _sc_scatter(src2d, dst2d, Ys, Wls)
    return _stage_d(S, Ys, degp, b1[None, :], Pp, Wl, W2, b2[:, None],
                    bl[None, :])
